# Initial kernel scaffold; baseline (speedup 1.0000x reference)
import functools

import jax
import jax.numpy as jnp
from jax import lax
from jax.experimental import pallas as pl
from jax.experimental.pallas import tpu as pltpu
from jax.experimental.pallas import tpu_sc as plsc

N_NODES = 10000
N_EDGES = 320000
N_GRAPHS = 64
DIM = 64
N_LAYERS = 5
BN_EPS = 1e-5

NC = 2
NS = 16
NW = NC * NS
CHUNK = 128
K = -(-N_EDGES // (NW * CHUNK))
EW = K * CHUNK
EPAD = NW * EW
ROWS_PER_TILE = 640
NACC = NS * ROWS_PER_TILE
ZROWS = 128
OUT_PER_TILE = N_NODES // NS


def _sc_agg_body(x_hbm, src_hbm, dst_hbm, out_hbm, src_v, dst_v, rows_v,
                 zbuf, acc, sem):
    c = lax.axis_index("c")
    s = lax.axis_index("s")
    wid = s * NC + c
    d = zbuf.shape[1]

    def zrow(i, _):
        for cc in range(d // 16):
            zbuf[i, pl.ds(cc * 16, 16)] = jnp.zeros((16,), jnp.float32)
        return 0
    lax.fori_loop(0, ZROWS, zrow, 0)
    for b in range(ROWS_PER_TILE // ZROWS):
        pltpu.sync_copy(zbuf, acc.at[pl.ds(s * ROWS_PER_TILE + b * ZROWS,
                                           ZROWS)])

    pltpu.sync_copy(src_hbm.at[pl.ds(wid * K, K)], src_v)
    pltpu.sync_copy(dst_hbm.at[pl.ds(wid * K, K)], dst_v)
    plsc.subcore_barrier()

    def chunk(j, _):
        pltpu.async_copy(x_hbm.at[src_v.at[j]], rows_v, sem).wait()
        pltpu.sync_copy(rows_v, acc.at[dst_v.at[j]], add=True)
        return 0
    lax.fori_loop(0, K, chunk, 0)
    plsc.subcore_barrier()

    pltpu.sync_copy(acc.at[pl.ds(s * OUT_PER_TILE, OUT_PER_TILE)],
                    out_hbm.at[c].at[pl.ds(s * OUT_PER_TILE, OUT_PER_TILE)])


@functools.lru_cache(maxsize=None)
def _make_sc_agg(d):
    mesh = plsc.VectorSubcoreMesh(core_axis_name="c", subcore_axis_name="s")
    return pl.kernel(
        _sc_agg_body,
        out_type=jax.ShapeDtypeStruct((NC, N_NODES, d), jnp.float32),
        mesh=mesh,
        scratch_types=[
            pltpu.VMEM((K, CHUNK), jnp.int32),
            pltpu.VMEM((K, CHUNK), jnp.int32),
            pltpu.VMEM((CHUNK, d), jnp.float32),
            pltpu.VMEM((ZROWS, d), jnp.float32),
            pltpu.VMEM_SHARED((NACC, d), jnp.float32),
            pltpu.SemaphoreType.DMA,
        ],
    )


def _tc_layer_body(x_ref, agg_ref, w1_ref, b1_ref, w2_ref, b2_ref,
                   g_ref, be_ref, gi_ref, h_ref, pool_ref):
    u = x_ref[...] + agg_ref[0] + agg_ref[1]
    t = jnp.dot(u, w1_ref[...], preferred_element_type=jnp.float32)
    t = jnp.maximum(t + b1_ref[...], 0.0)
    v = jnp.dot(t, w2_ref[...], preferred_element_type=jnp.float32)
    v = jnp.maximum(v + b2_ref[...], 0.0)
    mean = jnp.mean(v, axis=0, keepdims=True)
    var = jnp.mean(v * v, axis=0, keepdims=True) - mean * mean
    scale = g_ref[...] / jnp.sqrt(var + BN_EPS)
    hn = v * scale + (be_ref[...] - mean * scale)
    h_ref[...] = hn
    onehot = (lax.broadcasted_iota(jnp.int32, (N_GRAPHS, N_NODES), 0)
              == gi_ref[...]).astype(jnp.float32)
    pool_ref[...] = jnp.dot(onehot, hn, preferred_element_type=jnp.float32)


def _tc_layer(x, agg, p, gi2d):
    return pl.pallas_call(
        _tc_layer_body,
        out_shape=[
            jax.ShapeDtypeStruct((N_NODES, DIM), jnp.float32),
            jax.ShapeDtypeStruct((N_GRAPHS, DIM), jnp.float32),
        ],
    )(x, agg, p["W1"], p["b1"].reshape(1, -1), p["W2"],
      p["b2"].reshape(1, -1), p["gamma"].reshape(1, -1),
      p["beta"].reshape(1, -1), gi2d)


def kernel(node_features, edge_index, graph_index, params):
    pad = EPAD - N_EDGES
    src2d = jnp.concatenate(
        [edge_index[0], jnp.zeros((pad,), jnp.int32)]).reshape(NW * K, CHUNK)
    dst2d = jnp.concatenate(
        [edge_index[1], jnp.full((pad,), N_NODES, jnp.int32)]
    ).reshape(NW * K, CHUNK)
    gi2d = graph_index.reshape(1, N_NODES)

    x = node_features
    hs, pools = [], []
    for i in range(N_LAYERS):
        p = params[f"layer_{i}"]
        agg = _make_sc_agg(x.shape[1])(x, src2d, dst2d)
        h, pool = _tc_layer(x, agg, p, gi2d)
        x = h
        hs.append(h)
        pools.append(pool)
    return jnp.concatenate(pools, axis=1), jnp.concatenate(hs, axis=1)


# speedup vs baseline: 4.0469x; 4.0469x over previous
import functools

import jax
import jax.numpy as jnp
from jax import lax
from jax.experimental import pallas as pl
from jax.experimental.pallas import tpu as pltpu
from jax.experimental.pallas import tpu_sc as plsc

N_NODES = 10000
N_EDGES = 320000
N_GRAPHS = 64
DIM = 64
N_LAYERS = 5
BN_EPS = 1e-5

NC = 2
NS = 16
CHUNK = 128
K = 160
EW = K * CHUNK
EPAD = NS * EW
ROWS_PER_TILE = 640
NACC = NS * ROWS_PER_TILE
ZROWS = 128


def _sc_agg_body(x_hbm, src_hbm, dst_hbm, out_hbm, src_v, dst_v, rows_v,
                 zbuf, acc, sem):
    c = lax.axis_index("c")
    s = lax.axis_index("s")
    dh = zbuf.shape[1]

    def zrow(i, _):
        for cc in range(dh // 16):
            zbuf[i, pl.ds(cc * 16, 16)] = jnp.zeros((16,), jnp.float32)
        return 0
    lax.fori_loop(0, ZROWS, zrow, 0)
    for b in range(ROWS_PER_TILE // ZROWS):
        pltpu.sync_copy(zbuf, acc.at[pl.ds(s * ROWS_PER_TILE + b * ZROWS,
                                           ZROWS)])

    pltpu.sync_copy(src_hbm.at[pl.ds(s * K, K)], src_v)
    pltpu.sync_copy(dst_hbm.at[pl.ds(s * K, K)], dst_v)
    plsc.subcore_barrier()

    xh = x_hbm.at[c]

    def chunk(j, _):
        pltpu.async_copy(xh.at[src_v.at[j]], rows_v, sem).wait()
        pltpu.sync_copy(rows_v, acc.at[dst_v.at[j]], add=True)
        return 0
    lax.fori_loop(0, K, chunk, 0)
    plsc.subcore_barrier()

    pltpu.sync_copy(acc.at[pl.ds(s * ROWS_PER_TILE, ROWS_PER_TILE)],
                    out_hbm.at[c].at[pl.ds(s * ROWS_PER_TILE, ROWS_PER_TILE)])


@functools.lru_cache(maxsize=None)
def _make_sc_agg(d):
    dh = d // NC
    mesh = plsc.VectorSubcoreMesh(core_axis_name="c", subcore_axis_name="s")
    return pl.kernel(
        _sc_agg_body,
        out_type=jax.ShapeDtypeStruct((NC, NACC, dh), jnp.float32),
        mesh=mesh,
        compiler_params=pltpu.CompilerParams(use_tc_tiling_on_sc=False),
        scratch_types=[
            pltpu.VMEM((K, CHUNK), jnp.int32),
            pltpu.VMEM((K, CHUNK), jnp.int32),
            pltpu.VMEM((CHUNK, dh), jnp.float32),
            pltpu.VMEM((ZROWS, dh), jnp.float32),
            pltpu.VMEM_SHARED((NACC, dh), jnp.float32),
            pltpu.SemaphoreType.DMA,
        ],
    )


def _tc_layer_body(x_ref, agg_ref, w1_ref, b1_ref, w2_ref, b2_ref,
                   g_ref, be_ref, gi_ref, h_ref, pool_ref):
    agg = jnp.concatenate(
        [agg_ref[0, :N_NODES, :], agg_ref[1, :N_NODES, :]], axis=1)
    u = x_ref[...] + agg
    t = jnp.dot(u, w1_ref[...], preferred_element_type=jnp.float32)
    t = jnp.maximum(t + b1_ref[...], 0.0)
    v = jnp.dot(t, w2_ref[...], preferred_element_type=jnp.float32)
    v = jnp.maximum(v + b2_ref[...], 0.0)
    mean = jnp.mean(v, axis=0, keepdims=True)
    ctr = v - mean
    var = jnp.mean(ctr * ctr, axis=0, keepdims=True)
    hn = (v - mean) / jnp.sqrt(var + BN_EPS) * g_ref[...] + be_ref[...]
    h_ref[...] = hn
    onehot = (lax.broadcasted_iota(jnp.int32, (N_GRAPHS, N_NODES), 0)
              == gi_ref[...]).astype(jnp.float32)
    pool_ref[...] = jnp.dot(onehot, hn, preferred_element_type=jnp.float32,
                            precision=lax.Precision.HIGHEST)


def _tc_layer(x, agg, p, gi2d):
    return pl.pallas_call(
        _tc_layer_body,
        out_shape=[
            jax.ShapeDtypeStruct((N_NODES, DIM), jnp.float32),
            jax.ShapeDtypeStruct((N_GRAPHS, DIM), jnp.float32),
        ],
    )(x, agg, p["W1"], p["b1"].reshape(1, -1), p["W2"],
      p["b2"].reshape(1, -1), p["gamma"].reshape(1, -1),
      p["beta"].reshape(1, -1), gi2d)


def kernel(node_features, edge_index, graph_index, params):
    pad = EPAD - N_EDGES
    src2d = jnp.concatenate(
        [edge_index[0], jnp.zeros((pad,), jnp.int32)]).reshape(NS * K, CHUNK)
    dst2d = jnp.concatenate(
        [edge_index[1], jnp.full((pad,), N_NODES, jnp.int32)]
    ).reshape(NS * K, CHUNK)
    gi2d = graph_index.reshape(1, N_NODES)

    x = node_features
    hs, pools = [], []
    for i in range(N_LAYERS):
        p = params[f"layer_{i}"]
        d = x.shape[1]
        dh = d // NC
        xsplit = jnp.stack([x[:, :dh], x[:, dh:]])
        agg = _make_sc_agg(d)(xsplit, src2d, dst2d)
        h, pool = _tc_layer(x, agg, p, gi2d)
        x = h
        hs.append(h)
        pools.append(pool)
    return jnp.concatenate(pools, axis=1), jnp.concatenate(hs, axis=1)


# speedup vs baseline: 5.5355x; 1.3678x over previous
import functools

import jax
import jax.numpy as jnp
from jax import lax
from jax.experimental import pallas as pl
from jax.experimental.pallas import tpu as pltpu
from jax.experimental.pallas import tpu_sc as plsc

N_NODES = 10000
N_EDGES = 320000
N_GRAPHS = 64
DIM = 64
N_LAYERS = 5
BN_EPS = 1e-5

NC = 2
NS = 16
CHUNK = 128
K = 160
EW = K * CHUNK
EPAD = NS * EW
ROWS_PER_TILE = 640
NACC = NS * ROWS_PER_TILE
ZROWS = 128


NBUF = 4


def _sc_agg_body(x_hbm, src_hbm, dst_hbm, out_hbm, src_v, dst_v, rows_v,
                 zbuf, acc, gsem, ssem):
    c = lax.axis_index("c")
    s = lax.axis_index("s")
    dh = zbuf.shape[1]

    def zrow(i, _):
        for cc in range(dh // 16):
            zbuf[i, pl.ds(cc * 16, 16)] = jnp.zeros((16,), jnp.float32)
        return 0
    lax.fori_loop(0, ZROWS, zrow, 0)
    for b in range(ROWS_PER_TILE // ZROWS):
        pltpu.sync_copy(zbuf, acc.at[pl.ds(s * ROWS_PER_TILE + b * ZROWS,
                                           ZROWS)])

    pltpu.sync_copy(src_hbm.at[pl.ds(s * K, K)], src_v)
    pltpu.sync_copy(dst_hbm.at[pl.ds(s * K, K)], dst_v)
    plsc.subcore_barrier()

    xh = x_hbm.at[c]

    def group(g, _):
        base = g * NBUF
        for b in range(NBUF):
            j = base + b

            @pl.when(g > 0)
            def _():
                pltpu.make_async_copy(
                    rows_v.at[b], acc.at[dst_v.at[j - NBUF]],
                    ssem.at[b]).wait()
            pltpu.async_copy(xh.at[src_v.at[j]], rows_v.at[b], gsem.at[b])
        for b in range(NBUF):
            j = base + b
            pltpu.make_async_copy(xh.at[src_v.at[j]], rows_v.at[b],
                                  gsem.at[b]).wait()
            pltpu.async_copy(rows_v.at[b], acc.at[dst_v.at[j]], ssem.at[b],
                             add=True)
        return 0
    lax.fori_loop(0, K // NBUF, group, 0)
    for b in range(NBUF):
        pltpu.make_async_copy(rows_v.at[b], acc.at[dst_v.at[K - NBUF + b]],
                              ssem.at[b]).wait()
    plsc.subcore_barrier()

    pltpu.sync_copy(acc.at[pl.ds(s * ROWS_PER_TILE, ROWS_PER_TILE)],
                    out_hbm.at[c].at[pl.ds(s * ROWS_PER_TILE, ROWS_PER_TILE)])


@functools.lru_cache(maxsize=None)
def _make_sc_agg(d):
    dh = d // NC
    mesh = plsc.VectorSubcoreMesh(core_axis_name="c", subcore_axis_name="s")
    return pl.kernel(
        _sc_agg_body,
        out_type=jax.ShapeDtypeStruct((NC, NACC, dh), jnp.float32),
        mesh=mesh,
        compiler_params=pltpu.CompilerParams(use_tc_tiling_on_sc=False),
        scratch_types=[
            pltpu.VMEM((K, CHUNK), jnp.int32),
            pltpu.VMEM((K, CHUNK), jnp.int32),
            pltpu.VMEM((NBUF, CHUNK, dh), jnp.float32),
            pltpu.VMEM((ZROWS, dh), jnp.float32),
            pltpu.VMEM_SHARED((NACC, dh), jnp.float32),
            pltpu.SemaphoreType.DMA((NBUF,)),
            pltpu.SemaphoreType.DMA((NBUF,)),
        ],
    )


def _tc_layer_body(x_ref, agg_ref, w1_ref, b1_ref, w2_ref, b2_ref,
                   g_ref, be_ref, gi_ref, h_ref, pool_ref):
    agg = jnp.concatenate(
        [agg_ref[0, :N_NODES, :], agg_ref[1, :N_NODES, :]], axis=1)
    u = x_ref[...] + agg
    t = jnp.dot(u, w1_ref[...], preferred_element_type=jnp.float32)
    t = jnp.maximum(t + b1_ref[...], 0.0)
    v = jnp.dot(t, w2_ref[...], preferred_element_type=jnp.float32)
    v = jnp.maximum(v + b2_ref[...], 0.0)
    mean = jnp.mean(v, axis=0, keepdims=True)
    ctr = v - mean
    var = jnp.mean(ctr * ctr, axis=0, keepdims=True)
    hn = (v - mean) / jnp.sqrt(var + BN_EPS) * g_ref[...] + be_ref[...]
    h_ref[...] = hn
    onehot = (lax.broadcasted_iota(jnp.int32, (N_GRAPHS, N_NODES), 0)
              == gi_ref[...]).astype(jnp.float32)
    pool_ref[...] = jnp.dot(onehot, hn, preferred_element_type=jnp.float32,
                            precision=lax.Precision.HIGHEST)


def _tc_layer(x, agg, p, gi2d):
    return pl.pallas_call(
        _tc_layer_body,
        out_shape=[
            jax.ShapeDtypeStruct((N_NODES, DIM), jnp.float32),
            jax.ShapeDtypeStruct((N_GRAPHS, DIM), jnp.float32),
        ],
    )(x, agg, p["W1"], p["b1"].reshape(1, -1), p["W2"],
      p["b2"].reshape(1, -1), p["gamma"].reshape(1, -1),
      p["beta"].reshape(1, -1), gi2d)


def kernel(node_features, edge_index, graph_index, params):
    pad = EPAD - N_EDGES
    src2d = jnp.concatenate(
        [edge_index[0], jnp.zeros((pad,), jnp.int32)]).reshape(NS * K, CHUNK)
    dst2d = jnp.concatenate(
        [edge_index[1], jnp.full((pad,), N_NODES, jnp.int32)]
    ).reshape(NS * K, CHUNK)
    gi2d = graph_index.reshape(1, N_NODES)

    x = node_features
    hs, pools = [], []
    for i in range(N_LAYERS):
        p = params[f"layer_{i}"]
        d = x.shape[1]
        dh = d // NC
        xsplit = jnp.stack([x[:, :dh], x[:, dh:]])
        agg = _make_sc_agg(d)(xsplit, src2d, dst2d)
        h, pool = _tc_layer(x, agg, p, gi2d)
        x = h
        hs.append(h)
        pools.append(pool)
    return jnp.concatenate(pools, axis=1), jnp.concatenate(hs, axis=1)
